# baseline (speedup 1.0000x reference)
import jax
import jax.numpy as jnp
from jax.experimental import pallas as pl
from jax.experimental.pallas import tpu as pltpu

_SUB = 8


def _make_body(n_streams, sub):
    def body(x_ref, wih_ref, whh_ref, wtag_ref, btag_ref, out_ref,
             h_ref, c_ref, gx_ref, hs_ref):
        TC, B, E = x_ref.shape
        H = whh_ref.shape[0]
        TP = wtag_ref.shape[1]
        _SUB = sub
        NS = TC // _SUB
        BS = B // n_streams

        @pl.when(pl.program_id(1) == 0)
        def _reset_state():
            h_ref[...] = jnp.zeros_like(h_ref)
            c_ref[...] = jnp.zeros_like(c_ref)

        def project_inputs(j):
            xs = x_ref[pl.ds(j * _SUB, _SUB)]
            chan = jax.lax.broadcasted_iota(jnp.int32, xs.shape, 2)
            xs = jnp.where(chan < 20, xs, jax.nn.sigmoid(xs)).astype(
                jnp.bfloat16)
            ones = jnp.ones((_SUB, B, 8), jnp.bfloat16)
            xs = jnp.concatenate([xs, ones], axis=2)
            gx_ref[pl.ds(j * _SUB * B, _SUB * B), :] = jnp.dot(
                xs.reshape(_SUB * B, E + 8), wih_ref[...],
                preferred_element_type=jnp.float32)

        def project_tags(j):
            tags = (jnp.dot(hs_ref[pl.ds(j * _SUB * B, _SUB * B), :],
                            wtag_ref[...],
                            preferred_element_type=jnp.float32)
                    + btag_ref[...])
            out_ref[pl.ds(j * _SUB, _SUB)] = tags.reshape(_SUB, B, TP)

        whh = whh_ref[...]

        def step(t, lo, h, c):
            z = gx_ref[pl.ds(t * B + lo, BS), :] + jnp.dot(
                h, whh, preferred_element_type=jnp.float32)
            th = jnp.tanh(z)
            ti = th[:, 0 * H:1 * H]
            tf = th[:, 1 * H:2 * H]
            tg = th[:, 2 * H:3 * H]
            to = th[:, 3 * H:4 * H]
            c = ((tf + 1.0) * c + (ti + 1.0) * tg) * 0.5
            h = ((to + 1.0) * (0.5 * jnp.tanh(c))).astype(jnp.bfloat16)
            hs_ref[pl.ds(t * B + lo, BS), :] = h
            return h, c

        hh = [h_ref[pl.ds(s * BS, BS), :] for s in range(n_streams)]
        cc = [c_ref[pl.ds(s * BS, BS), :] for s in range(n_streams)]

        project_inputs(0)
        for j in range(NS):
            if j + 1 < NS:
                project_inputs(j + 1)
            for ts in range(_SUB):
                t = j * _SUB + ts
                for s in range(n_streams):
                    hh[s], cc[s] = step(t, s * BS, hh[s], cc[s])
            if j >= 1:
                project_tags(j - 1)
        project_tags(NS - 1)

        for s in range(n_streams):
            h_ref[pl.ds(s * BS, BS), :] = hh[s]
            c_ref[pl.ds(s * BS, BS), :] = cc[s]

    return body


def kernel(x, w_ih, w_hh, b_ih, b_hh, w_tag, b_tag):
    S, B, E = x.shape
    H = w_ih.shape[0] // 4
    T = w_tag.shape[0]

    B_pad = -(-B // 8) * 8
    T_pad = -(-T // 128) * 128

    b_block = B_pad // 2 if B_pad % 16 == 0 else B_pad
    n_streams = 1

    t_chunk = min(64, S)
    while S % t_chunk:
        t_chunk //= 2
    sub = min(_SUB, t_chunk)
    assert t_chunk % sub == 0
    S_pad = pl.cdiv(S, t_chunk) * t_chunk

    x_p = jnp.pad(x, ((0, S_pad - S), (0, B_pad - B), (0, 0)))

    gcol = jnp.arange(4 * H)
    gate_scale = jnp.where((gcol >= 2 * H) & (gcol < 3 * H), 1.0, 0.5)

    wih_rows = jnp.concatenate(
        [jnp.transpose(w_ih) * gate_scale,
         ((b_ih + b_hh) * gate_scale).reshape(1, 4 * H),
         jnp.zeros((7, 4 * H), jnp.float32)], axis=0)
    wih_t = wih_rows.astype(jnp.bfloat16)
    whh_t = (jnp.transpose(w_hh) * gate_scale).astype(jnp.bfloat16)
    wtag_t = jnp.pad(jnp.transpose(w_tag).astype(jnp.bfloat16),
                     ((0, 0), (0, T_pad - T)))
    btag = jnp.pad(b_tag, (0, T_pad - T)).reshape(1, T_pad)

    out = pl.pallas_call(
        _make_body(n_streams, sub),
        out_shape=jax.ShapeDtypeStruct((S_pad, B_pad, T_pad), jnp.float32),
        grid=(B_pad // b_block, S_pad // t_chunk),
        in_specs=[
            pl.BlockSpec((t_chunk, b_block, E), lambda b, t: (t, b, 0)),
            pl.BlockSpec((E + 8, 4 * H), lambda b, t: (0, 0)),
            pl.BlockSpec((H, 4 * H), lambda b, t: (0, 0)),
            pl.BlockSpec((H, T_pad), lambda b, t: (0, 0)),
            pl.BlockSpec((1, T_pad), lambda b, t: (0, 0)),
        ],
        out_specs=pl.BlockSpec((t_chunk, b_block, T_pad),
                               lambda b, t: (t, b, 0)),
        scratch_shapes=[
            pltpu.VMEM((b_block, H), jnp.bfloat16),
            pltpu.VMEM((b_block, H), jnp.float32),
            pltpu.VMEM((t_chunk * b_block, 4 * H), jnp.float32),
            pltpu.VMEM((t_chunk * b_block, H), jnp.bfloat16),
        ],
        compiler_params=pltpu.CompilerParams(
            dimension_semantics=("parallel", "arbitrary"),
            vmem_limit_bytes=56 * 1024 * 1024),
    )(x_p, wih_t, whh_t, wtag_t, btag)
    return out[:S, :B, :T]
